# baseline (speedup 1.0000x reference)
import functools

import jax
import jax.numpy as jnp
from jax import lax
from jax.experimental import pallas as pl
from jax.experimental.pallas import tpu as pltpu
from jax.experimental.pallas import tpu_sc as plsc

MAXLEN = 200
EMBED = 64
LANES = 16
BBLK = 128
NGBUF = 4
NOBUF = 2


def kernel(inputs, token_table, pos_table):
    B, L = inputs.shape
    NC, NS = 2, 16
    NW = NC * NS
    assert B == NW * BBLK and L == MAXLEN
    idx_t = inputs.T.astype(jnp.int32)
    table2 = jnp.pad(token_table, ((0, 0), (0, EMBED)))

    mesh = plsc.VectorSubcoreMesh(
        core_axis_name="c", subcore_axis_name="s", num_cores=NC, num_subcores=NS
    )

    @functools.partial(
        pl.kernel,
        out_type=jax.ShapeDtypeStruct((MAXLEN, 8, B // BBLK, EMBED // 8, BBLK), jnp.float32),
        mesh=mesh,
        scratch_types=[
            pltpu.VMEM((MAXLEN, BBLK), jnp.int32),
            pltpu.VMEM((MAXLEN, EMBED), jnp.float32),
            pltpu.VMEM((NGBUF, BBLK, 2 * EMBED), jnp.float32),
            pltpu.VMEM((NOBUF, 8, EMBED // 8, BBLK), jnp.float32),
            pltpu.SemaphoreType.DMA((NGBUF,)),
            pltpu.SemaphoreType.DMA((NOBUF,)),
        ],
        compiler_params=pltpu.CompilerParams(
            use_tc_tiling_on_sc=False, needs_layout_passes=False
        ),
    )
    def run(
        idx_hbm, table_hbm, pos_hbm, out_hbm, idx_v, pos_v, gbuf, obuf, gsem, ssem
    ):
        wid = lax.axis_index("s") * NC + lax.axis_index("c")
        col = wid * BBLK
        pltpu.sync_copy(pos_hbm, pos_v)
        pltpu.sync_copy(idx_hbm.at[:, pl.ds(col, BBLK)], idx_v)

        def gather_dst(g):
            return gbuf.at[g]

        def issue_gather(t, g):
            pltpu.async_copy(table_hbm.at[idx_v.at[t]], gather_dst(g), gsem.at[g])

        def wait_gather(t, g):
            pltpu.make_async_copy(
                table_hbm.at[idx_v.at[t]], gather_dst(g), gsem.at[g]
            ).wait()

        def out_slice(t):
            return out_hbm.at[t, :, wid, :, :]

        def store_src(o):
            return obuf.at[o]

        def wait_store(t, o):
            pltpu.make_async_copy(store_src(o), out_slice(t), ssem.at[o]).wait()

        iota16 = lax.iota(jnp.int32, 16)

        for g in range(NGBUF - 1):
            issue_gather(g, g)

        def outer(i, carry):
            for g in range(NGBUF):
                t = i * NGBUF + g
                o = t % NOBUF
                wait_gather(t, g)

                @pl.when(t >= NOBUF)
                def _():
                    wait_store(t - NOBUF, o)

                prow = [
                    pos_v[t, pl.ds(j * LANES, LANES)]
                    for j in range(EMBED // LANES)
                ]
                @plsc.parallel_loop(0, BBLK // LANES, unroll=1)
                def _(k):
                    sl = pl.ds(k * LANES, LANES)
                    bvec = iota16 + k * LANES
                    for j in range(EMBED // LANES):
                        for e_sub in range(LANES):
                            e = j * LANES + e_sub
                            obuf[o, e >> 3, e & 7, sl] = jnp.full(
                                (LANES,), prow[j][e_sub]
                            )

                    def dgrp(i3, c):
                        for d_sub in range(4):
                            rot = (iota16 + (i3 * 4 + d_sub)) & (LANES - 1)
                            for j in range(EMBED // LANES):
                                evec = rot + j * LANES
                                v = plsc.load_gather(gbuf.at[g], [bvec, evec])
                                plsc.addupdate_scatter(
                                    obuf.at[o], [evec >> 3, evec & 7, bvec], v
                                )
                        return c

                    lax.fori_loop(0, 4, dgrp, 0)
                pltpu.async_copy(store_src(o), out_slice(t), ssem.at[o])

                gp = (g + NGBUF - 1) % NGBUF

                @pl.when(t + NGBUF - 1 < MAXLEN)
                def _():
                    issue_gather(t + NGBUF - 1, gp)
            return carry

        lax.fori_loop(0, MAXLEN // NGBUF, outer, 0)

        for o in range(NOBUF):
            wait_store(MAXLEN - NOBUF + o, o)

    out = run(idx_t, table2, pos_table)
    out = jnp.transpose(out, (0, 1, 3, 2, 4)).reshape(MAXLEN, EMBED, B)
    return jnp.transpose(out, (2, 0, 1))
